# baseline (speedup 1.0000x reference)
import functools

import jax
import jax.numpy as jnp
from jax import lax
from jax.experimental import pallas as pl
from jax.experimental.pallas import tpu as pltpu
from jax.experimental.pallas import tpu_sc as plsc

_NW = 32
_G = 128
_K = 10
_NBUF = 2


@functools.lru_cache(maxsize=None)
def _build_gather(n, dim):
    n_per_w = n // _NW
    c_rows = _K * _G
    nchunks = n_per_w // c_rows
    assert n_per_w % c_rows == 0 and nchunks % _NBUF == 0
    mesh = plsc.VectorSubcoreMesh(core_axis_name="c", subcore_axis_name="s")

    @functools.partial(
        pl.kernel,
        mesh=mesh,
        out_type=jax.ShapeDtypeStruct((n, dim), jnp.float32),
        scratch_types=[
            pltpu.VMEM((n_per_w,), jnp.int32),
            pltpu.VMEM((_NBUF, c_rows, dim), jnp.float32),
            pltpu.SemaphoreType.DMA((_NBUF, _K)),
            pltpu.SemaphoreType.DMA((_NBUF,)),
        ],
        compiler_params=pltpu.CompilerParams(use_tc_tiling_on_sc=False),
    )
    def gather_kernel(idx_hbm, table_hbm, out_hbm, idx_v, rows_v, gsem, osem):
        wid = lax.axis_index("s") * 2 + lax.axis_index("c")
        base = wid * n_per_w
        pltpu.sync_copy(idx_hbm.at[pl.ds(base, n_per_w)], idx_v)

        def g_descs(c, b):
            return [
                pltpu.make_async_copy(
                    table_hbm.at[idx_v.at[pl.ds(c * c_rows + j * _G, _G)]],
                    rows_v.at[b, pl.ds(j * _G, _G)],
                    gsem.at[b, j],
                )
                for j in range(_K)
            ]

        def o_desc(c, b):
            return pltpu.make_async_copy(
                rows_v.at[b],
                out_hbm.at[pl.ds(base + c * c_rows, c_rows)],
                osem.at[b],
            )

        def fire(c, b):
            for d in g_descs(c, b):
                d.start()

        def drain(c, b):
            for d in g_descs(c, b):
                d.wait()

        for b in range(_NBUF):
            fire(b, b)

        def body(t, carry):
            for b in range(_NBUF):
                c = t * _NBUF + b
                drain(c, b)
                o_desc(c, b).start()
                o_desc(c, b).wait()
                fire(c + _NBUF, b)
            return carry

        lax.fori_loop(0, nchunks // _NBUF - 1, body, 0)

        for b in range(_NBUF):
            c = nchunks - _NBUF + b
            drain(c, b)
            o_desc(c, b).start()
            o_desc(c, b).wait()

    return gather_kernel


def kernel(x, planes_weight):
    b, s = x.shape
    _, dim = planes_weight.shape
    n = b * s
    idx = x.reshape(n).astype(jnp.int32)
    out = _build_gather(n, dim)(idx, planes_weight)
    return out.reshape(b, s, dim)


# speedup vs baseline: 1.7431x; 1.7431x over previous
import functools

import jax
import jax.numpy as jnp
from jax import lax
from jax.experimental import pallas as pl
from jax.experimental.pallas import tpu as pltpu
from jax.experimental.pallas import tpu_sc as plsc

_NW = 32
_G = 128
_K = 10
_NBUF = 2


@functools.lru_cache(maxsize=None)
def _build_gather(n, dim):
    n_per_w = n // _NW
    c_rows = _K * _G
    nchunks = n_per_w // c_rows
    assert n_per_w % c_rows == 0 and nchunks % _NBUF == 0
    mesh = plsc.VectorSubcoreMesh(core_axis_name="c", subcore_axis_name="s")

    @functools.partial(
        pl.kernel,
        mesh=mesh,
        out_type=jax.ShapeDtypeStruct((n, dim), jnp.float32),
        scratch_types=[
            pltpu.VMEM((n_per_w,), jnp.int32),
            pltpu.VMEM((_NBUF, c_rows, dim), jnp.float32),
            pltpu.SemaphoreType.DMA((_NBUF, _K)),
            pltpu.SemaphoreType.DMA((_NBUF,)),
        ],
        compiler_params=pltpu.CompilerParams(use_tc_tiling_on_sc=False),
    )
    def gather_kernel(idx_hbm, table_hbm, out_hbm, idx_v, rows_v, gsem, osem):
        wid = lax.axis_index("s") * 2 + lax.axis_index("c")
        base = wid * n_per_w
        pltpu.sync_copy(idx_hbm.at[pl.ds(base, n_per_w)], idx_v)

        def g_descs(c, b):
            return [
                pltpu.make_async_copy(
                    table_hbm.at[idx_v.at[pl.ds(c * c_rows + j * _G, _G)]],
                    rows_v.at[b, pl.ds(j * _G, _G)],
                    gsem.at[b, j],
                )
                for j in range(_K)
            ]

        def o_desc(c, b):
            return pltpu.make_async_copy(
                rows_v.at[b],
                out_hbm.at[pl.ds(base + c * c_rows, c_rows)],
                osem.at[b],
            )

        def fire(c, b):
            for d in g_descs(c, b):
                d.start()

        def drain(c, b):
            for d in g_descs(c, b):
                d.wait()

        for b in range(_NBUF):
            fire(b, b)

        def body(t, carry):
            for b in range(_NBUF):
                c = t * _NBUF + b
                drain(c, b)
                o_desc(c, b).start()
                o_desc(c, b).wait()
                fire(c + _NBUF, b)
            return carry

        lax.fori_loop(0, nchunks // _NBUF - 1, body, 0)

        for b in range(_NBUF):
            c = nchunks - _NBUF + b
            drain(c, b)
            o_desc(c, b).start()
            o_desc(c, b).wait()

    return gather_kernel


def kernel(x, planes_weight):
    b, s = x.shape
    _, dim = planes_weight.shape
    n = b * s
    idx = x.T.reshape(n).astype(jnp.int32)
    out = _build_gather(n, dim)(idx, planes_weight)
    return out.reshape(s, b, dim).transpose(1, 0, 2)
